# Initial kernel scaffold; baseline (speedup 1.0000x reference)
import jax
import jax.numpy as jnp
from jax.experimental import pallas as pl


def kernel(hidden_states, top_k_indices, top_k_weights, gate_up_proj, down_proj):
    raise NotImplementedError("write your pallas kernel here")


# speedup vs baseline: 2.3660x; 2.3660x over previous
import jax
import jax.numpy as jnp
from jax.experimental import pallas as pl
from jax.experimental.pallas import tpu as pltpu

_FF = 512


def _moe_body(idx_ref, wgt_ref, hs_ref, gup_ref, down_ref, out_ref):
    e = pl.program_id(0)
    w = jnp.sum(jnp.where(idx_ref[...] == e, wgt_ref[...], 0.0), axis=1)

    hs = hs_ref[...]
    gu = jax.lax.dot_general(
        hs, gup_ref[0], (((1,), (1,)), ((), ())),
        preferred_element_type=jnp.float32)
    gate = gu[:, :_FF]
    up = gu[:, _FF:]
    act = gate * jax.nn.sigmoid(gate) * up
    eo = jax.lax.dot_general(
        act, down_ref[0], (((1,), (1,)), ((), ())),
        preferred_element_type=jnp.float32)
    contrib = eo * w[:, None]

    @pl.when(e == 0)
    def _init():
        out_ref[...] = contrib

    @pl.when(e != 0)
    def _acc():
        out_ref[...] += contrib


def kernel(hidden_states, top_k_indices, top_k_weights, gate_up_proj, down_proj):
    T, H = hidden_states.shape
    E, FF2, _ = gate_up_proj.shape
    K = top_k_indices.shape[1]

    return pl.pallas_call(
        _moe_body,
        grid=(E,),
        in_specs=[
            pl.BlockSpec((T, K), lambda e: (0, 0)),
            pl.BlockSpec((T, K), lambda e: (0, 0)),
            pl.BlockSpec((T, H), lambda e: (0, 0)),
            pl.BlockSpec((1, FF2, H), lambda e: (e, 0, 0)),
            pl.BlockSpec((1, H, FF2 // 2), lambda e: (e, 0, 0)),
        ],
        out_specs=pl.BlockSpec((T, H), lambda e: (0, 0)),
        out_shape=jax.ShapeDtypeStruct((T, H), jnp.float32),
        compiler_params=pltpu.CompilerParams(
            dimension_semantics=("arbitrary",),
        ),
    )(top_k_indices, top_k_weights, hidden_states, gate_up_proj, down_proj)


# speedup vs baseline: 2.6056x; 1.1013x over previous
import jax
import jax.numpy as jnp
from jax.experimental import pallas as pl
from jax.experimental.pallas import tpu as pltpu

_FF = 512


_EPB = 2


def _moe_body(idx_ref, wgt_ref, hs_ref, gup_ref, down_ref, out_ref):
    g = pl.program_id(0)
    hs = hs_ref[...]
    contrib = jnp.zeros_like(out_ref)
    for i in range(_EPB):
        e = g * _EPB + i
        w = jnp.sum(jnp.where(idx_ref[...] == e, wgt_ref[...], 0.0), axis=1)
        gu = jax.lax.dot_general(
            hs, gup_ref[i], (((1,), (1,)), ((), ())),
            preferred_element_type=jnp.float32)
        gate = gu[:, :_FF]
        up = gu[:, _FF:]
        act = gate * jax.nn.sigmoid(gate) * up
        eo = jax.lax.dot_general(
            act, down_ref[i], (((1,), (1,)), ((), ())),
            preferred_element_type=jnp.float32)
        contrib = contrib + eo * w[:, None]

    @pl.when(g == 0)
    def _init():
        out_ref[...] = contrib

    @pl.when(g != 0)
    def _acc():
        out_ref[...] += contrib


def kernel(hidden_states, top_k_indices, top_k_weights, gate_up_proj, down_proj):
    T, H = hidden_states.shape
    E, FF2, _ = gate_up_proj.shape
    K = top_k_indices.shape[1]

    return pl.pallas_call(
        _moe_body,
        grid=(E // _EPB,),
        in_specs=[
            pl.BlockSpec((T, K), lambda e: (0, 0)),
            pl.BlockSpec((T, K), lambda e: (0, 0)),
            pl.BlockSpec((T, H), lambda e: (0, 0)),
            pl.BlockSpec((_EPB, FF2, H), lambda e: (e, 0, 0)),
            pl.BlockSpec((_EPB, H, FF2 // 2), lambda e: (e, 0, 0)),
        ],
        out_specs=pl.BlockSpec((T, H), lambda e: (0, 0)),
        out_shape=jax.ShapeDtypeStruct((T, H), jnp.float32),
        compiler_params=pltpu.CompilerParams(
            dimension_semantics=("arbitrary",),
        ),
    )(top_k_indices, top_k_weights, hidden_states, gate_up_proj, down_proj)
